# baseline (speedup 1.0000x reference)
import functools

import jax
import jax.numpy as jnp
from jax import lax
from jax.experimental import pallas as pl
from jax.experimental.pallas import tpu as pltpu

N_NODES = 10000
N_EDGES = 320000
EMB = 128
HID = 2 * EMB
NGRAPH = 64
NLAYER = 5


def _bn_cols(t, g, b):
    m = jnp.mean(t, axis=0, keepdims=True)
    v = jnp.mean((t - m) ** 2, axis=0, keepdims=True)
    return (t - m) / jnp.sqrt(v + 1e-5) * g + b


def _atom_body(x_ref, w1_ref, b1_ref, g_ref, bb_ref, w2_ref, b2_ref, vn0_ref,
               out_ref):
    x = x_ref[...]
    t = jnp.dot(x, w1_ref[...], preferred_element_type=jnp.float32) + b1_ref[...]
    t = jnp.maximum(_bn_cols(t, g_ref[...], bb_ref[...]), 0.0)
    h = jnp.dot(t, w2_ref[...], preferred_element_type=jnp.float32) + b2_ref[...]
    out_ref[...] = h + vn0_ref[...]


def _atom_encode(x_p, atom_W1_p, atom_b1, atom_bn_g, atom_bn_b, atom_W2,
                 atom_b2, vn0):
    return pl.pallas_call(
        _atom_body,
        out_shape=jax.ShapeDtypeStruct((N_NODES, EMB), jnp.float32),
    )(x_p, atom_W1_p, atom_b1.reshape(1, EMB), atom_bn_g.reshape(1, EMB),
      atom_bn_b.reshape(1, EMB), atom_W2, atom_b2.reshape(1, EMB), vn0)


def _emb_body(ea_ref, w_ref, b_ref, out_ref):
    out_ref[...] = (
        jnp.dot(ea_ref[...], w_ref[...], preferred_element_type=jnp.float32)
        + b_ref[...])


def _edge_emb(edge_attr_p, w_p, b):
    blk = 8000
    return pl.pallas_call(
        _emb_body,
        grid=(N_EDGES // blk,),
        in_specs=[
            pl.BlockSpec((blk, 16), lambda i: (i, 0)),
            pl.BlockSpec((16, EMB), lambda i: (0, 0)),
            pl.BlockSpec((1, EMB), lambda i: (0, 0)),
        ],
        out_specs=pl.BlockSpec((blk, EMB), lambda i: (i, 0)),
        out_shape=jax.ShapeDtypeStruct((N_EDGES, EMB), jnp.float32),
    )(edge_attr_p, w_p, b.reshape(1, EMB))


def _layer_body(last, h_in_ref, p0_ref, p1_ref, batch_ref, vn_ref, eps_ref,
                gw1_ref, gb1_ref, gg1_ref, gbb1_ref, gw2_ref, gb2_ref,
                bg_ref, bb_ref, out_h_ref, out_vn_ref=None):
    h_in = h_in_ref[...]
    aggr = p0_ref[...] + p1_ref[...]
    z = (1.0 + eps_ref[0, 0]) * h_in + aggr
    t1 = jnp.dot(z, gw1_ref[...], preferred_element_type=jnp.float32) + gb1_ref[...]
    t1 = jnp.maximum(_bn_cols(t1, gg1_ref[...], gbb1_ref[...]), 0.0)
    z2 = jnp.dot(t1, gw2_ref[...], preferred_element_type=jnp.float32) + gb2_ref[...]
    hc = _bn_cols(z2, bg_ref[...], bb_ref[...])
    if last:
        out_h_ref[...] = hc
        return
    hc = jnp.maximum(hc, 0.0)
    oh = (batch_ref[...] == lax.broadcasted_iota(jnp.int32, (N_NODES, NGRAPH), 1)
          ).astype(jnp.float32)
    vt = lax.dot_general(oh, h_in, (((0,), (0,)), ((), ())),
                         preferred_element_type=jnp.float32,
                         precision=lax.Precision.HIGHEST) + vn_ref[...]
    out_vn_ref[...] = vt
    out_h_ref[...] = hc


def _layer_dense(last, h_in, p0, p1, batch2d, vn, eps_i, gw1, gb1, gg1, gbb1,
                 gw2, gb2, bg, bb):
    if last:
        out_shape = jax.ShapeDtypeStruct((N_NODES, EMB), jnp.float32)
    else:
        out_shape = (jax.ShapeDtypeStruct((N_NODES, EMB), jnp.float32),
                     jax.ShapeDtypeStruct((NGRAPH, EMB), jnp.float32))
    return pl.pallas_call(
        functools.partial(_layer_body, last),
        out_shape=out_shape,
    )(h_in, p0, p1, batch2d, vn, eps_i.reshape(1, 1),
      gw1, gb1.reshape(1, HID), gg1.reshape(1, HID), gbb1.reshape(1, HID),
      gw2, gb2.reshape(1, EMB), bg.reshape(1, EMB), bb.reshape(1, EMB))


def kernel(x, edge_index, edge_attr, batch, atom_W1, atom_b1, atom_bn_g,
           atom_bn_b, atom_W2, atom_b2, vn_emb, bond_W, bond_b, eps, gin_W1,
           gin_b1, gin_bn_g, gin_bn_b, gin_W2, gin_b2, bn_g, bn_b, vn_W1,
           vn_b1, vn_bn1_g, vn_bn1_b, vn_W2, vn_b2, vn_bn2_g, vn_bn2_b):
    src = edge_index[0]
    dst = edge_index[1]
    x_p = jnp.pad(x, ((0, 0), (0, 128 - x.shape[1])))
    atom_W1_p = jnp.pad(atom_W1, ((0, 128 - atom_W1.shape[0]), (0, 0)))
    edge_attr_p = jnp.pad(edge_attr, ((0, 0), (0, 16 - edge_attr.shape[1])))
    bond_W_p = jnp.pad(bond_W, ((0, 0), (0, 16 - bond_W.shape[1]), (0, 0)))
    batch2d = batch.reshape(N_NODES, 1)

    def _bn_ref(h, g, b):
        m = jnp.mean(h, axis=0)
        v = jnp.var(h, axis=0)
        return (h - m) / jnp.sqrt(v + 1e-5) * g + b

    h = jax.nn.relu(_bn_ref(x @ atom_W1 + atom_b1, atom_bn_g, atom_bn_b))
    h = h @ atom_W2 + atom_b2
    vn = jnp.broadcast_to(vn_emb[0], (NGRAPH, EMB))

    zeros_p = jnp.zeros((N_NODES, EMB), jnp.float32)
    h_out = None
    for layer in range(NLAYER):
        h_in = h + vn[batch]
        emb = _edge_emb(edge_attr_p, bond_W_p[layer], bond_b[layer])
        msg = jax.nn.relu(h_in[src] + emb)
        aggr = jax.ops.segment_sum(msg, dst, num_segments=N_NODES)
        last = layer == NLAYER - 1
        if layer < 3:
            z = (1.0 + eps[layer]) * h_in + aggr
            t1n = jax.nn.relu(_bn_ref(z @ gin_W1[layer] + gin_b1[layer],
                                      gin_bn_g[layer], gin_bn_b[layer]))
            z2 = t1n @ gin_W2[layer] + gin_b2[layer]
            hc = jax.nn.relu(_bn_ref(z2, bn_g[layer], bn_b[layer]))
            vt = jax.ops.segment_sum(h_in, batch, num_segments=NGRAPH) + vn
        elif last:
            h_out = _layer_dense(True, h_in, aggr, zeros_p, batch2d, vn,
                                 eps[layer], gin_W1[layer], gin_b1[layer],
                                 gin_bn_g[layer], gin_bn_b[layer],
                                 gin_W2[layer], gin_b2[layer], bn_g[layer],
                                 bn_b[layer])
            break
        else:
            hc, vt = _layer_dense(False, h_in, aggr, zeros_p, batch2d, vn,
                                  eps[layer], gin_W1[layer], gin_b1[layer],
                                  gin_bn_g[layer], gin_bn_b[layer],
                                  gin_W2[layer], gin_b2[layer], bn_g[layer],
                                  bn_b[layer])
        t = jax.nn.relu(_bn_ref(vt @ vn_W1[layer] + vn_b1[layer],
                                vn_bn1_g[layer], vn_bn1_b[layer]))
        s = t @ vn_W2[layer] + vn_b2[layer]
        vn = jax.nn.relu(_bn_ref(s, vn_bn2_g[layer], vn_bn2_b[layer]))
        h = hc
    return h_out


# speedup vs baseline: 1.2992x; 1.2992x over previous
import functools

import jax
import jax.numpy as jnp
from jax import lax
from jax.experimental import pallas as pl
from jax.experimental.pallas import tpu as pltpu
from jax.experimental.pallas import tpu_sc as plsc

N_NODES = 10000
N_EDGES = 320000
EMB = 128
HID = 2 * EMB
NGRAPH = 64
NLAYER = 5


def _bn_cols(t, g, b):
    m = jnp.mean(t, axis=0, keepdims=True)
    v = jnp.mean((t - m) ** 2, axis=0, keepdims=True)
    return (t - m) / jnp.sqrt(v + 1e-5) * g + b


def _atom_body(x_ref, w1_ref, b1_ref, g_ref, bb_ref, w2_ref, b2_ref, vn0_ref,
               out_ref):
    x = x_ref[...]
    t = jnp.dot(x, w1_ref[...], preferred_element_type=jnp.float32) + b1_ref[...]
    t = jnp.maximum(_bn_cols(t, g_ref[...], bb_ref[...]), 0.0)
    h = jnp.dot(t, w2_ref[...], preferred_element_type=jnp.float32) + b2_ref[...]
    out_ref[...] = h + vn0_ref[...]


def _atom_encode(x_p, atom_W1_p, atom_b1, atom_bn_g, atom_bn_b, atom_W2,
                 atom_b2, vn0):
    return pl.pallas_call(
        _atom_body,
        out_shape=jax.ShapeDtypeStruct((N_NODES, EMB), jnp.float32),
    )(x_p, atom_W1_p, atom_b1.reshape(1, EMB), atom_bn_g.reshape(1, EMB),
      atom_bn_b.reshape(1, EMB), atom_W2, atom_b2.reshape(1, EMB), vn0)


def _emb_body(ea_ref, w_ref, b_ref, out_ref):
    out_ref[...] = (
        jnp.dot(ea_ref[...], w_ref[...], preferred_element_type=jnp.float32)
        + b_ref[...])


def _edge_emb(edge_attr_p, w_p, b):
    blk = 8000
    return pl.pallas_call(
        _emb_body,
        grid=(N_EDGES // blk,),
        in_specs=[
            pl.BlockSpec((blk, 16), lambda i: (i, 0)),
            pl.BlockSpec((16, EMB), lambda i: (0, 0)),
            pl.BlockSpec((1, EMB), lambda i: (0, 0)),
        ],
        out_specs=pl.BlockSpec((blk, EMB), lambda i: (i, 0)),
        out_shape=jax.ShapeDtypeStruct((N_EDGES, EMB), jnp.float32),
    )(edge_attr_p, w_p, b.reshape(1, EMB))


_NW = 32
_EPW = N_EDGES // _NW
_EC = 80
_NCH = _EPW // _EC
_RPT = N_NODES // 16


def _edge_sc_body(h_hbm, emb_hbm, src_hbm, dst_hbm, out_hbm,
                  acc, src_v, dst_v, rows_v, emb_v, zbuf, sem):
    c = lax.axis_index("c")
    s = lax.axis_index("s")
    wid = s * 2 + c

    def zb(i, _):
        for k8 in range(8):
            zbuf[i, pl.ds(k8 * 16, 16)] = jnp.zeros((16,), jnp.float32)
        return 0
    lax.fori_loop(0, 104, zb, 0)
    for j in range(6):
        pltpu.sync_copy(zbuf, acc.at[pl.ds(s * 624 + j * 104, 104)])

    @pl.when(s == 0)
    def _():
        pltpu.sync_copy(zbuf.at[pl.ds(0, 16)], acc.at[pl.ds(9984, 16)])
    plsc.subcore_barrier()

    base0 = wid * _EPW

    def chunk(i, _):
        base = base0 + i * _EC
        pltpu.sync_copy(src_hbm.at[pl.ds(base, _EC)], src_v)
        pltpu.sync_copy(dst_hbm.at[pl.ds(base, _EC)], dst_v)
        pltpu.async_copy(h_hbm.at[src_v], rows_v, sem).wait()
        pltpu.sync_copy(emb_hbm.at[pl.ds(base, _EC)], emb_v)

        def erow(e, _):
            for k8 in range(8):
                sl = pl.ds(k8 * 16, 16)
                rows_v[e, sl] = jnp.maximum(rows_v[e, sl] + emb_v[e, sl], 0.0)
            return 0
        lax.fori_loop(0, _EC, erow, 0)
        pltpu.sync_copy(rows_v, acc.at[dst_v], add=True)
        return 0
    lax.fori_loop(0, _NCH, chunk, 0)
    plsc.subcore_barrier()
    pltpu.sync_copy(acc.at[pl.ds(s * 624, 624)],
                    out_hbm.at[c, pl.ds(s * 624, 624)])

    @pl.when(s == 0)
    def _():
        pltpu.sync_copy(acc.at[pl.ds(9984, 16)],
                        out_hbm.at[c, pl.ds(9984, 16)])


def _edge_sc(h_in, emb, src, dst):
    mesh = plsc.VectorSubcoreMesh(core_axis_name="c", subcore_axis_name="s")
    return pl.kernel(
        _edge_sc_body,
        mesh=mesh,
        out_type=jax.ShapeDtypeStruct((2, N_NODES, EMB), jnp.float32),
        scratch_types=[
            pltpu.VMEM_SHARED((N_NODES, EMB), jnp.float32),
            pltpu.VMEM((_EC,), jnp.int32),
            pltpu.VMEM((_EC,), jnp.int32),
            pltpu.VMEM((_EC, EMB), jnp.float32),
            pltpu.VMEM((_EC, EMB), jnp.float32),
            pltpu.VMEM((104, EMB), jnp.float32),
            pltpu.SemaphoreType.DMA,
        ],
    )(h_in, emb, src, dst)


def _layer_body(last, h_in_ref, p0_ref, p1_ref, batch_ref, vn_ref, eps_ref,
                gw1_ref, gb1_ref, gg1_ref, gbb1_ref, gw2_ref, gb2_ref,
                bg_ref, bb_ref, out_h_ref, out_vn_ref=None):
    h_in = h_in_ref[...]
    aggr = p0_ref[...] + p1_ref[...]
    z = (1.0 + eps_ref[0, 0]) * h_in + aggr
    t1 = jnp.dot(z, gw1_ref[...], preferred_element_type=jnp.float32) + gb1_ref[...]
    t1 = jnp.maximum(_bn_cols(t1, gg1_ref[...], gbb1_ref[...]), 0.0)
    z2 = jnp.dot(t1, gw2_ref[...], preferred_element_type=jnp.float32) + gb2_ref[...]
    hc = _bn_cols(z2, bg_ref[...], bb_ref[...])
    if last:
        out_h_ref[...] = hc
        return
    hc = jnp.maximum(hc, 0.0)
    oh = (batch_ref[...] == lax.broadcasted_iota(jnp.int32, (N_NODES, NGRAPH), 1)
          ).astype(jnp.float32)
    vt = lax.dot_general(oh, h_in, (((0,), (0,)), ((), ())),
                         preferred_element_type=jnp.float32,
                         precision=lax.Precision.HIGHEST) + vn_ref[...]
    out_vn_ref[...] = vt
    out_h_ref[...] = hc


def _layer_dense(last, h_in, p0, p1, batch2d, vn, eps_i, gw1, gb1, gg1, gbb1,
                 gw2, gb2, bg, bb):
    if last:
        out_shape = jax.ShapeDtypeStruct((N_NODES, EMB), jnp.float32)
    else:
        out_shape = (jax.ShapeDtypeStruct((N_NODES, EMB), jnp.float32),
                     jax.ShapeDtypeStruct((NGRAPH, EMB), jnp.float32))
    return pl.pallas_call(
        functools.partial(_layer_body, last),
        out_shape=out_shape,
    )(h_in, p0, p1, batch2d, vn, eps_i.reshape(1, 1),
      gw1, gb1.reshape(1, HID), gg1.reshape(1, HID), gbb1.reshape(1, HID),
      gw2, gb2.reshape(1, EMB), bg.reshape(1, EMB), bb.reshape(1, EMB))


def kernel(x, edge_index, edge_attr, batch, atom_W1, atom_b1, atom_bn_g,
           atom_bn_b, atom_W2, atom_b2, vn_emb, bond_W, bond_b, eps, gin_W1,
           gin_b1, gin_bn_g, gin_bn_b, gin_W2, gin_b2, bn_g, bn_b, vn_W1,
           vn_b1, vn_bn1_g, vn_bn1_b, vn_W2, vn_b2, vn_bn2_g, vn_bn2_b):
    src = edge_index[0]
    dst = edge_index[1]
    x_p = jnp.pad(x, ((0, 0), (0, 128 - x.shape[1])))
    atom_W1_p = jnp.pad(atom_W1, ((0, 128 - atom_W1.shape[0]), (0, 0)))
    edge_attr_p = jnp.pad(edge_attr, ((0, 0), (0, 16 - edge_attr.shape[1])))
    bond_W_p = jnp.pad(bond_W, ((0, 0), (0, 16 - bond_W.shape[1]), (0, 0)))
    batch2d = batch.reshape(N_NODES, 1)

    def _bn_ref(h, g, b):
        m = jnp.mean(h, axis=0)
        v = jnp.var(h, axis=0)
        return (h - m) / jnp.sqrt(v + 1e-5) * g + b

    h = jax.nn.relu(_bn_ref(x @ atom_W1 + atom_b1, atom_bn_g, atom_bn_b))
    h = h @ atom_W2 + atom_b2
    vn = jnp.broadcast_to(vn_emb[0], (NGRAPH, EMB))

    zeros_p = jnp.zeros((N_NODES, EMB), jnp.float32)
    h_out = None
    for layer in range(NLAYER):
        h_in = h + vn[batch]
        emb = _edge_emb(edge_attr_p, bond_W_p[layer], bond_b[layer])
        last = layer == NLAYER - 1
        if layer < 3:
            msg = jax.nn.relu(h_in[src] + emb)
            aggr = jax.ops.segment_sum(msg, dst, num_segments=N_NODES)
            z = (1.0 + eps[layer]) * h_in + aggr
            t1n = jax.nn.relu(_bn_ref(z @ gin_W1[layer] + gin_b1[layer],
                                      gin_bn_g[layer], gin_bn_b[layer]))
            z2 = t1n @ gin_W2[layer] + gin_b2[layer]
            hc = jax.nn.relu(_bn_ref(z2, bn_g[layer], bn_b[layer]))
            vt = jax.ops.segment_sum(h_in, batch, num_segments=NGRAPH) + vn
        elif last:
            part = _edge_sc(h_in, emb, src, dst)
            h_out = _layer_dense(True, h_in, part[0], part[1], batch2d, vn,
                                 eps[layer], gin_W1[layer], gin_b1[layer],
                                 gin_bn_g[layer], gin_bn_b[layer],
                                 gin_W2[layer], gin_b2[layer], bn_g[layer],
                                 bn_b[layer])
            break
        else:
            part = _edge_sc(h_in, emb, src, dst)
            hc, vt = _layer_dense(False, h_in, part[0], part[1], batch2d, vn,
                                  eps[layer], gin_W1[layer], gin_b1[layer],
                                  gin_bn_g[layer], gin_bn_b[layer],
                                  gin_W2[layer], gin_b2[layer], bn_g[layer],
                                  bn_b[layer])
        t = jax.nn.relu(_bn_ref(vt @ vn_W1[layer] + vn_b1[layer],
                                vn_bn1_g[layer], vn_bn1_b[layer]))
        s = t @ vn_W2[layer] + vn_b2[layer]
        vn = jax.nn.relu(_bn_ref(s, vn_bn2_g[layer], vn_bn2_b[layer]))
        h = hc
    return h_out


# speedup vs baseline: 1.2995x; 1.0003x over previous
import functools

import jax
import jax.numpy as jnp
from jax import lax
from jax.experimental import pallas as pl
from jax.experimental.pallas import tpu as pltpu
from jax.experimental.pallas import tpu_sc as plsc

N_NODES = 10000
N_EDGES = 320000
EMB = 128
HID = 2 * EMB
NGRAPH = 64
NLAYER = 5


def _bn_cols(t, g, b):
    m = jnp.mean(t, axis=0, keepdims=True)
    v = jnp.mean((t - m) ** 2, axis=0, keepdims=True)
    return (t - m) / jnp.sqrt(v + 1e-5) * g + b


def _emb_body(ea_ref, w_ref, b_ref, out_ref):
    out_ref[...] = (
        jnp.dot(ea_ref[...], w_ref[...], preferred_element_type=jnp.float32)
        + b_ref[...])


def _edge_emb(edge_attr_p, w_p, b):
    blk = 8000
    return pl.pallas_call(
        _emb_body,
        grid=(N_EDGES // blk,),
        in_specs=[
            pl.BlockSpec((blk, 16), lambda i: (i, 0)),
            pl.BlockSpec((16, EMB), lambda i: (0, 0)),
            pl.BlockSpec((1, EMB), lambda i: (0, 0)),
        ],
        out_specs=pl.BlockSpec((blk, EMB), lambda i: (i, 0)),
        out_shape=jax.ShapeDtypeStruct((N_EDGES, EMB), jnp.float32),
    )(edge_attr_p, w_p, b.reshape(1, EMB))


_NW = 32
_EPW = N_EDGES // _NW
_EC = 80
_NCH = _EPW // _EC
_RPT = N_NODES // 16


def _edge_sc_body(h_hbm, emb_hbm, src_hbm, dst_hbm, out_hbm,
                  acc, src_v, dst_v, rows_v, emb_v, zbuf, sem):
    c = lax.axis_index("c")
    s = lax.axis_index("s")
    wid = s * 2 + c

    def zb(i, _):
        for k8 in range(8):
            zbuf[i, pl.ds(k8 * 16, 16)] = jnp.zeros((16,), jnp.float32)
        return 0
    lax.fori_loop(0, 104, zb, 0)
    for j in range(6):
        pltpu.sync_copy(zbuf, acc.at[pl.ds(s * 624 + j * 104, 104)])

    @pl.when(s == 0)
    def _():
        pltpu.sync_copy(zbuf.at[pl.ds(0, 16)], acc.at[pl.ds(9984, 16)])
    plsc.subcore_barrier()

    base0 = wid * _EPW

    def chunk(i, _):
        base = base0 + i * _EC
        pltpu.sync_copy(src_hbm.at[pl.ds(base, _EC)], src_v)
        pltpu.sync_copy(dst_hbm.at[pl.ds(base, _EC)], dst_v)
        pltpu.async_copy(h_hbm.at[src_v], rows_v, sem).wait()
        pltpu.sync_copy(emb_hbm.at[pl.ds(base, _EC)], emb_v)

        def erow(e, _):
            for k8 in range(8):
                sl = pl.ds(k8 * 16, 16)
                rows_v[e, sl] = jnp.maximum(rows_v[e, sl] + emb_v[e, sl], 0.0)
            return 0
        lax.fori_loop(0, _EC, erow, 0)
        pltpu.sync_copy(rows_v, acc.at[dst_v], add=True)
        return 0
    lax.fori_loop(0, _NCH, chunk, 0)
    plsc.subcore_barrier()
    pltpu.sync_copy(acc.at[pl.ds(s * 624, 624)],
                    out_hbm.at[c, pl.ds(s * 624, 624)])

    @pl.when(s == 0)
    def _():
        pltpu.sync_copy(acc.at[pl.ds(9984, 16)],
                        out_hbm.at[c, pl.ds(9984, 16)])


def _edge_sc(h_in, emb, src, dst):
    mesh = plsc.VectorSubcoreMesh(core_axis_name="c", subcore_axis_name="s")
    return pl.kernel(
        _edge_sc_body,
        mesh=mesh,
        out_type=jax.ShapeDtypeStruct((2, N_NODES, EMB), jnp.float32),
        scratch_types=[
            pltpu.VMEM_SHARED((N_NODES, EMB), jnp.float32),
            pltpu.VMEM((_EC,), jnp.int32),
            pltpu.VMEM((_EC,), jnp.int32),
            pltpu.VMEM((_EC, EMB), jnp.float32),
            pltpu.VMEM((_EC, EMB), jnp.float32),
            pltpu.VMEM((104, EMB), jnp.float32),
            pltpu.SemaphoreType.DMA,
        ],
    )(h_in, emb, src, dst)


def _layer_body(last, h_in_ref, p0_ref, p1_ref, batch_ref, vn_ref, eps_ref,
                gw1_ref, gb1_ref, gg1_ref, gbb1_ref, gw2_ref, gb2_ref,
                bg_ref, bb_ref, out_h_ref, out_vn_ref=None):
    h_in = h_in_ref[...]
    aggr = p0_ref[...] + p1_ref[...]
    z = (1.0 + eps_ref[0, 0]) * h_in + aggr
    t1 = jnp.dot(z, gw1_ref[...], preferred_element_type=jnp.float32) + gb1_ref[...]
    t1 = jnp.maximum(_bn_cols(t1, gg1_ref[...], gbb1_ref[...]), 0.0)
    z2 = jnp.dot(t1, gw2_ref[...], preferred_element_type=jnp.float32) + gb2_ref[...]
    hc = _bn_cols(z2, bg_ref[...], bb_ref[...])
    if last:
        out_h_ref[...] = hc
        return
    hc = jnp.maximum(hc, 0.0)
    oh = (batch_ref[...] == lax.broadcasted_iota(jnp.int32, (N_NODES, NGRAPH), 1)
          ).astype(jnp.float32)
    vt = lax.dot_general(oh, h_in, (((0,), (0,)), ((), ())),
                         preferred_element_type=jnp.float32,
                         precision=lax.Precision.HIGHEST) + vn_ref[...]
    out_vn_ref[...] = vt
    out_h_ref[...] = hc


def _layer_dense(last, h_in, p0, p1, batch2d, vn, eps_i, gw1, gb1, gg1, gbb1,
                 gw2, gb2, bg, bb):
    if last:
        out_shape = jax.ShapeDtypeStruct((N_NODES, EMB), jnp.float32)
    else:
        out_shape = (jax.ShapeDtypeStruct((N_NODES, EMB), jnp.float32),
                     jax.ShapeDtypeStruct((NGRAPH, EMB), jnp.float32))
    return pl.pallas_call(
        functools.partial(_layer_body, last),
        out_shape=out_shape,
    )(h_in, p0, p1, batch2d, vn, eps_i.reshape(1, 1),
      gw1, gb1.reshape(1, HID), gg1.reshape(1, HID), gbb1.reshape(1, HID),
      gw2, gb2.reshape(1, EMB), bg.reshape(1, EMB), bb.reshape(1, EMB))


def kernel(x, edge_index, edge_attr, batch, atom_W1, atom_b1, atom_bn_g,
           atom_bn_b, atom_W2, atom_b2, vn_emb, bond_W, bond_b, eps, gin_W1,
           gin_b1, gin_bn_g, gin_bn_b, gin_W2, gin_b2, bn_g, bn_b, vn_W1,
           vn_b1, vn_bn1_g, vn_bn1_b, vn_W2, vn_b2, vn_bn2_g, vn_bn2_b):
    src = edge_index[0]
    dst = edge_index[1]
    edge_attr_p = jnp.pad(edge_attr, ((0, 0), (0, 16 - edge_attr.shape[1])))
    bond_W_p = jnp.pad(bond_W, ((0, 0), (0, 16 - bond_W.shape[1]), (0, 0)))
    batch2d = batch.reshape(N_NODES, 1)

    def _bn_ref(h, g, b):
        m = jnp.mean(h, axis=0)
        v = jnp.var(h, axis=0)
        return (h - m) / jnp.sqrt(v + 1e-5) * g + b

    h = jax.nn.relu(_bn_ref(x @ atom_W1 + atom_b1, atom_bn_g, atom_bn_b))
    h = h @ atom_W2 + atom_b2
    vn = jnp.broadcast_to(vn_emb[0], (NGRAPH, EMB))

    h_out = None
    for layer in range(NLAYER):
        h_in = h + vn[batch]
        emb = _edge_emb(edge_attr_p, bond_W_p[layer], bond_b[layer])
        last = layer == NLAYER - 1
        if layer < 3:
            msg = jax.nn.relu(h_in[src] + emb)
            aggr = jax.ops.segment_sum(msg, dst, num_segments=N_NODES)
            z = (1.0 + eps[layer]) * h_in + aggr
            t1n = jax.nn.relu(_bn_ref(z @ gin_W1[layer] + gin_b1[layer],
                                      gin_bn_g[layer], gin_bn_b[layer]))
            z2 = t1n @ gin_W2[layer] + gin_b2[layer]
            hc = jax.nn.relu(_bn_ref(z2, bn_g[layer], bn_b[layer]))
            vt = jax.ops.segment_sum(h_in, batch, num_segments=NGRAPH) + vn
        elif last:
            part = _edge_sc(h_in, emb, src, dst)
            h_out = _layer_dense(True, h_in, part[0], part[1], batch2d, vn,
                                 eps[layer], gin_W1[layer], gin_b1[layer],
                                 gin_bn_g[layer], gin_bn_b[layer],
                                 gin_W2[layer], gin_b2[layer], bn_g[layer],
                                 bn_b[layer])
            break
        else:
            part = _edge_sc(h_in, emb, src, dst)
            hc, vt = _layer_dense(False, h_in, part[0], part[1], batch2d, vn,
                                  eps[layer], gin_W1[layer], gin_b1[layer],
                                  gin_bn_g[layer], gin_bn_b[layer],
                                  gin_W2[layer], gin_b2[layer], bn_g[layer],
                                  bn_b[layer])
        t = jax.nn.relu(_bn_ref(vt @ vn_W1[layer] + vn_b1[layer],
                                vn_bn1_g[layer], vn_bn1_b[layer]))
        s = t @ vn_W2[layer] + vn_b2[layer]
        vn = jax.nn.relu(_bn_ref(s, vn_bn2_g[layer], vn_bn2_b[layer]))
        h = hc
    return h_out
